# baseline (speedup 1.0000x reference)
import functools

import jax
import jax.numpy as jnp
from jax import lax
from jax.experimental import pallas as pl
from jax.experimental.pallas import tpu as pltpu
from jax.experimental.pallas import tpu_sc as plsc


@functools.lru_cache(maxsize=None)
def _build_gather(vocab, dim, n_rows):
    info = plsc.get_sparse_core_info()
    nc, ns = info.num_cores, info.num_subcores
    nw = nc * ns
    rows_per_w = n_rows // nw
    chunk = 16
    n_chunks = rows_per_w // chunk
    nb = 3

    mesh = plsc.VectorSubcoreMesh(core_axis_name="c", subcore_axis_name="s")

    @functools.partial(
        pl.kernel,
        mesh=mesh,
        out_type=jax.ShapeDtypeStruct((n_rows, dim), jnp.float32),
        scratch_types=[
            pltpu.VMEM((n_chunks, chunk), jnp.int32),
        ]
        + [pltpu.VMEM((chunk, dim), jnp.float32) for _ in range(nb)]
        + [pltpu.SemaphoreType.DMA for _ in range(2 * nb)],
    )
    def gather_kernel(idx_hbm, table_hbm, out_hbm, idx_v, *rest):
        bufs = rest[:nb]
        gsems = rest[nb:2 * nb]
        ssems = rest[2 * nb:]
        wid = lax.axis_index("s") * nc + lax.axis_index("c")
        base = wid * rows_per_w
        pltpu.sync_copy(idx_hbm.at[wid], idx_v)

        g_handles = [None] * n_chunks
        s_handles = [None] * n_chunks
        for j in range(min(nb, n_chunks)):
            g_handles[j] = pltpu.async_copy(
                table_hbm.at[idx_v.at[j]], bufs[j], gsems[j])
        for i in range(n_chunks):
            if i >= 1 and i + nb - 1 < n_chunks:
                s_handles[i - 1].wait()
                j = i + nb - 1
                g_handles[j] = pltpu.async_copy(
                    table_hbm.at[idx_v.at[j]], bufs[j % nb], gsems[j % nb])
            g_handles[i].wait()
            s_handles[i] = pltpu.async_copy(
                bufs[i % nb], out_hbm.at[pl.ds(base + i * chunk, chunk)],
                ssems[i % nb])
        for i in range(max(0, n_chunks - nb), n_chunks):
            s_handles[i].wait()

    return gather_kernel, nw, n_chunks, chunk


@functools.lru_cache(maxsize=None)
def _build_tc_gather(vocab, dim, n_rows):
    r_per_step = 8
    n_steps = n_rows // r_per_step

    def body(ids_ref, *refs):
        row_refs = refs[:r_per_step]
        o_ref = refs[r_per_step]
        for j in range(r_per_step):
            o_ref[j, :] = row_refs[j][0, 0, :]

    def make_spec(j):
        return pl.BlockSpec(
            (1, 1, dim), lambda i, ids: (ids[i * r_per_step + j], 0, 0))

    return pl.pallas_call(
        body,
        grid_spec=pltpu.PrefetchScalarGridSpec(
            num_scalar_prefetch=1,
            grid=(n_steps,),
            in_specs=[make_spec(j) for j in range(r_per_step)],
            out_specs=pl.BlockSpec((r_per_step, dim), lambda i, ids: (i, 0)),
        ),
        out_shape=jax.ShapeDtypeStruct((n_rows, dim), jnp.float32),
    )


def kernel(input_ids, token_embedding):
    b, s = input_ids.shape
    vocab, dim = token_embedding.shape
    n_rows = b * s
    n_tc = n_rows // 4
    n_sc = n_rows - n_tc
    ids_flat = input_ids.reshape(n_rows)
    fn, nw, n_chunks, chunk = _build_gather(vocab, dim, n_sc)
    idx = ids_flat[:n_sc].reshape(nw, n_chunks, chunk)
    out_sc = fn(idx, token_embedding)
    tc_fn = _build_tc_gather(vocab, dim, n_tc)
    table3 = token_embedding.reshape(vocab, 1, dim)
    out_tc = tc_fn(ids_flat[n_sc:], *([table3] * 8))
    out = jnp.concatenate([out_sc, out_tc], axis=0)
    return out.reshape(b, s, dim)


# speedup vs baseline: 12.7475x; 12.7475x over previous
import functools

import jax
import jax.numpy as jnp
from jax import lax
from jax.experimental import pallas as pl
from jax.experimental.pallas import tpu as pltpu
from jax.experimental.pallas import tpu_sc as plsc


@functools.lru_cache(maxsize=None)
def _build_gather(vocab, dim, n_rows):
    info = plsc.get_sparse_core_info()
    nc, ns = info.num_cores, info.num_subcores
    nw = nc * ns
    rows_per_w = n_rows // nw
    chunk = 16
    n_chunks = rows_per_w // chunk
    nb = 3

    mesh = plsc.VectorSubcoreMesh(core_axis_name="c", subcore_axis_name="s")

    @functools.partial(
        pl.kernel,
        mesh=mesh,
        out_type=jax.ShapeDtypeStruct((n_rows, dim), jnp.float32),
        scratch_types=[
            pltpu.VMEM((n_chunks, chunk), jnp.int32),
        ]
        + [pltpu.VMEM((chunk, dim), jnp.float32) for _ in range(nb)]
        + [pltpu.SemaphoreType.DMA for _ in range(2 * nb)],
    )
    def gather_kernel(idx_hbm, table_hbm, out_hbm, idx_v, *rest):
        bufs = rest[:nb]
        gsems = rest[nb:2 * nb]
        ssems = rest[2 * nb:]
        wid = lax.axis_index("s") * nc + lax.axis_index("c")
        base = wid * rows_per_w
        pltpu.sync_copy(idx_hbm.at[wid], idx_v)

        g_handles = [None] * n_chunks
        s_handles = [None] * n_chunks
        for j in range(min(nb, n_chunks)):
            g_handles[j] = pltpu.async_copy(
                table_hbm.at[idx_v.at[j]], bufs[j], gsems[j])
        for i in range(n_chunks):
            if i >= 1 and i + nb - 1 < n_chunks:
                s_handles[i - 1].wait()
                j = i + nb - 1
                g_handles[j] = pltpu.async_copy(
                    table_hbm.at[idx_v.at[j]], bufs[j % nb], gsems[j % nb])
            g_handles[i].wait()
            s_handles[i] = pltpu.async_copy(
                bufs[i % nb], out_hbm.at[pl.ds(base + i * chunk, chunk)],
                ssems[i % nb])
        for i in range(max(0, n_chunks - nb), n_chunks):
            s_handles[i].wait()

    return gather_kernel, nw, n_chunks, chunk


def kernel(input_ids, token_embedding):
    b, s = input_ids.shape
    vocab, dim = token_embedding.shape
    n_rows = b * s
    fn, nw, n_chunks, chunk = _build_gather(vocab, dim, n_rows)
    idx = input_ids.reshape(nw, n_chunks, chunk)
    out = fn(idx, token_embedding)
    return out.reshape(b, s, dim)
